# baseline (speedup 1.0000x reference)
import jax
import jax.numpy as jnp
from jax.experimental import pallas as pl


BM = 512


def _tok_kernel(x_ref, w0_ref, b0_ref, w1_ref, b1_ref, w2_ref, b2_ref,
                w3_ref, b3_ref, cb_ref, sem_ref):
    f32 = jnp.float32
    bf16 = jnp.bfloat16

    def dot(a, b):
        return jnp.dot(a.astype(bf16), b.astype(bf16),
                       preferred_element_type=f32)

    h = x_ref[...]
    for w_ref, b_ref in ((w0_ref, b0_ref), (w1_ref, b1_ref), (w2_ref, b2_ref)):
        h = dot(h, w_ref[...]) + b_ref[...]
        h = h * jax.lax.logistic(h)
    res = dot(h, w3_ref[...]) + b3_ref[...]

    K = cb_ref.shape[1]
    n_levels = cb_ref.shape[0]
    iota_k = jax.lax.broadcasted_iota(jnp.int32, (BM, K), 1)
    for l in range(n_levels):
        cb = cb_ref[l]
        cb_sq = jnp.sum(cb * cb, axis=-1)[None, :]
        dist = -2.0 * dot(res, cb.T) + cb_sq
        min_val = jnp.min(dist, axis=-1, keepdims=True)
        eq = dist == min_val
        idx = jnp.min(jnp.where(eq, iota_k, K), axis=-1)
        sem_ref[0, l, :] = idx
        if l + 1 < n_levels:
            cb_hi = cb.astype(bf16)
            r1 = cb - cb_hi.astype(f32)
            cb_mid = r1.astype(bf16)
            cb_lo = (r1 - cb_mid.astype(f32)).astype(bf16)
            oh = (iota_k == idx[:, None]).astype(bf16)
            picked = (dot(oh, cb_hi) + dot(oh, cb_mid)) + dot(oh, cb_lo)
            res = res - picked


def kernel(x, ids, user_ids, seq_mask, W0, b0, W1, b1, W2, b2, W3, b3,
           codebooks):
    b, n, d_in = x.shape
    m = b * n
    grid = m // BM
    L = codebooks.shape[0]
    bf16 = jnp.bfloat16
    xf = x.reshape(m, d_in)

    full = lambda shape: pl.BlockSpec(shape, lambda i: (0,) * len(shape))
    sem = pl.pallas_call(
        _tok_kernel,
        grid=(grid,),
        in_specs=[
            pl.BlockSpec((BM, d_in), lambda i: (i, 0)),
            full(W0.shape), full((1, b0.shape[0])),
            full(W1.shape), full((1, b1.shape[0])),
            full(W2.shape), full((1, b2.shape[0])),
            full(W3.shape), full((1, b3.shape[0])),
            full(codebooks.shape),
        ],
        out_specs=pl.BlockSpec((1, L, BM), lambda i: (i, 0, 0)),
        out_shape=jax.ShapeDtypeStruct((grid, L, BM), jnp.int32),
    )(xf, W0.astype(bf16), b0.reshape(1, -1), W1.astype(bf16),
      b1.reshape(1, -1), W2.astype(bf16), b2.reshape(1, -1),
      W3.astype(bf16), b3.reshape(1, -1), codebooks)

    sem_ids = sem.transpose(0, 2, 1).reshape(m, L)
    token_type_ids = jnp.tile(jnp.arange(L, dtype=jnp.int32), n)
    return (user_ids, sem_ids, token_type_ids)


# speedup vs baseline: 1.0850x; 1.0850x over previous
import jax
import jax.numpy as jnp
from jax.experimental import pallas as pl


BM = 1024


def _tok_kernel(x_ref, w0_ref, b0_ref, w1_ref, b1_ref, w2_ref, b2_ref,
                w3_ref, b3_ref, cb_ref, sem_ref):
    f32 = jnp.float32
    bf16 = jnp.bfloat16

    def dot(a, b):
        return jnp.dot(a.astype(bf16), b.astype(bf16),
                       preferred_element_type=f32)

    h = x_ref[...]
    for w_ref, b_ref in ((w0_ref, b0_ref), (w1_ref, b1_ref), (w2_ref, b2_ref)):
        h = dot(h, w_ref[...]) + b_ref[...]
        h = h * jax.lax.logistic(h)
    res = dot(h, w3_ref[...]) + b3_ref[...]

    K = cb_ref.shape[1]
    n_levels = cb_ref.shape[0]
    iota_k = jax.lax.broadcasted_iota(jnp.int32, (BM, K), 1)
    for l in range(n_levels):
        cb = cb_ref[l]
        cb_sq = jnp.sum(cb * cb, axis=-1)[None, :]
        dist = -2.0 * dot(res, cb.T) + cb_sq
        min_val = jnp.min(dist, axis=-1, keepdims=True)
        eq = dist == min_val
        idx = jnp.min(jnp.where(eq, iota_k, K), axis=-1)
        sem_ref[0, l, :] = idx
        if l + 1 < n_levels:
            cb_hi = cb.astype(bf16)
            r1 = cb - cb_hi.astype(f32)
            cb_mid = r1.astype(bf16)
            cb_lo = (r1 - cb_mid.astype(f32)).astype(bf16)
            oh = (iota_k == idx[:, None]).astype(bf16)
            picked = (dot(oh, cb_hi) + dot(oh, cb_mid)) + dot(oh, cb_lo)
            res = res - picked


def kernel(x, ids, user_ids, seq_mask, W0, b0, W1, b1, W2, b2, W3, b3,
           codebooks):
    b, n, d_in = x.shape
    m = b * n
    grid = m // BM
    L = codebooks.shape[0]
    bf16 = jnp.bfloat16
    xf = x.reshape(m, d_in).astype(bf16)

    full = lambda shape: pl.BlockSpec(shape, lambda i: (0,) * len(shape))
    sem = pl.pallas_call(
        _tok_kernel,
        grid=(grid,),
        in_specs=[
            pl.BlockSpec((BM, d_in), lambda i: (i, 0)),
            full(W0.shape), full((1, b0.shape[0])),
            full(W1.shape), full((1, b1.shape[0])),
            full(W2.shape), full((1, b2.shape[0])),
            full(W3.shape), full((1, b3.shape[0])),
            full(codebooks.shape),
        ],
        out_specs=pl.BlockSpec((1, L, BM), lambda i: (i, 0, 0)),
        out_shape=jax.ShapeDtypeStruct((grid, L, BM), jnp.int32),
    )(xf, W0.astype(bf16), b0.reshape(1, -1), W1.astype(bf16),
      b1.reshape(1, -1), W2.astype(bf16), b2.reshape(1, -1),
      W3.astype(bf16), b3.reshape(1, -1), codebooks)

    sem_ids = sem.transpose(0, 2, 1).reshape(m, L)
    token_type_ids = jnp.tile(jnp.arange(L, dtype=jnp.int32), n)
    return (user_ids, sem_ids, token_type_ids)


# speedup vs baseline: 1.1030x; 1.0166x over previous
import jax
import jax.numpy as jnp
from jax.experimental import pallas as pl


BM = 2048


def _tok_kernel(x_ref, w0_ref, b0_ref, w1_ref, b1_ref, w2_ref, b2_ref,
                w3_ref, b3_ref, cb_ref, sem_ref):
    f32 = jnp.float32
    bf16 = jnp.bfloat16

    def dot(a, b):
        return jnp.dot(a.astype(bf16), b.astype(bf16),
                       preferred_element_type=f32)

    h = x_ref[...]
    for w_ref, b_ref in ((w0_ref, b0_ref), (w1_ref, b1_ref), (w2_ref, b2_ref)):
        h = dot(h, w_ref[...]) + b_ref[...]
        h = h * jax.lax.logistic(h)
    res = dot(h, w3_ref[...]) + b3_ref[...]

    K = cb_ref.shape[1]
    n_levels = cb_ref.shape[0]
    iota_k = jax.lax.broadcasted_iota(jnp.int32, (BM, K), 1)
    for l in range(n_levels):
        cb = cb_ref[l]
        cb_sq = jnp.sum(cb * cb, axis=-1)[None, :]
        dist = -2.0 * dot(res, cb.T) + cb_sq
        min_val = jnp.min(dist, axis=-1, keepdims=True)
        eq = dist == min_val
        idx = jnp.min(jnp.where(eq, iota_k, K), axis=-1)
        sem_ref[0, l, :] = idx
        if l + 1 < n_levels:
            cb_hi = cb.astype(bf16)
            r1 = cb - cb_hi.astype(f32)
            cb_mid = r1.astype(bf16)
            cb_lo = (r1 - cb_mid.astype(f32)).astype(bf16)
            oh = (iota_k == idx[:, None]).astype(bf16)
            picked = (dot(oh, cb_hi) + dot(oh, cb_mid)) + dot(oh, cb_lo)
            res = res - picked


def kernel(x, ids, user_ids, seq_mask, W0, b0, W1, b1, W2, b2, W3, b3,
           codebooks):
    b, n, d_in = x.shape
    m = b * n
    grid = m // BM
    L = codebooks.shape[0]
    bf16 = jnp.bfloat16
    xf = x.reshape(m, d_in).astype(bf16)

    full = lambda shape: pl.BlockSpec(shape, lambda i: (0,) * len(shape))
    sem = pl.pallas_call(
        _tok_kernel,
        grid=(grid,),
        in_specs=[
            pl.BlockSpec((BM, d_in), lambda i: (i, 0)),
            full(W0.shape), full((1, b0.shape[0])),
            full(W1.shape), full((1, b1.shape[0])),
            full(W2.shape), full((1, b2.shape[0])),
            full(W3.shape), full((1, b3.shape[0])),
            full(codebooks.shape),
        ],
        out_specs=pl.BlockSpec((1, L, BM), lambda i: (i, 0, 0)),
        out_shape=jax.ShapeDtypeStruct((grid, L, BM), jnp.int32),
    )(xf, W0.astype(bf16), b0.reshape(1, -1), W1.astype(bf16),
      b1.reshape(1, -1), W2.astype(bf16), b2.reshape(1, -1),
      W3.astype(bf16), b3.reshape(1, -1), codebooks)

    sem_ids = sem.transpose(0, 2, 1).reshape(m, L)
    token_type_ids = jnp.tile(jnp.arange(L, dtype=jnp.int32), n)
    return (user_ids, sem_ids, token_type_ids)
